# Initial kernel scaffold; baseline (speedup 1.0000x reference)
import jax
import jax.numpy as jnp
from jax.experimental import pallas as pl


def kernel(belief, probability, payoff, mistrust, neighbors):
    raise NotImplementedError("write your pallas kernel here")


# speedup vs baseline: 82.6354x; 82.6354x over previous
import functools

import jax
import jax.numpy as jnp
from jax import lax
from jax.experimental import pallas as pl
from jax.experimental.pallas import tpu as pltpu
from jax.experimental.pallas import tpu_sc as plsc

NW = 32
L = 16
B = 448
W = 16
ICH = 128


def _log_tables_kernel(p_ref, lp_ref, l1p_ref):
    p = p_ref[...]
    lp_ref[...] = jnp.log(p)
    l1p_ref[...] = jnp.log(1.0 - p)


def _make_sc_combine(n_pad, deg, c_per_w, blks):
    mesh = plsc.VectorSubcoreMesh(
        core_axis_name="c", subcore_axis_name="s", num_cores=2,
        num_subcores=16)
    grp_per_blk = B // L
    nch = B * deg // ICH

    @functools.partial(
        pl.kernel,
        out_type=jax.ShapeDtypeStruct((n_pad,), jnp.float32),
        mesh=mesh,
        scratch_types=[
            pltpu.VMEM((nch, ICH), jnp.int32),
            pltpu.VMEM((B * deg, W), jnp.float32),
            pltpu.VMEM((B,), jnp.float32),
            pltpu.VMEM((B,), jnp.float32),
            pltpu.VMEM((B,), jnp.float32),
            pltpu.VMEM((B,), jnp.float32),
            pltpu.VMEM((B,), jnp.float32),
            pltpu.SemaphoreType.DMA,
        ],
        compiler_params=pltpu.CompilerParams(
            needs_layout_passes=False, use_tc_tiling_on_sc=False),
    )
    def sc_combine(tbl, nbr, prior0, lp, l1p, mist, out,
                   idx_v, mail_v, prior_v, lp_v, l1p_v, mist_v, out_v, sem):
        wid = lax.axis_index("s") * 2 + lax.axis_index("c")
        base = wid * c_per_w
        lanes = lax.iota(jnp.int32, 16)
        col_b = jnp.full((16,), 0, jnp.int32)
        col_s = jnp.full((16,), 1, jnp.int32)
        col_t = jnp.full((16,), 2, jnp.int32)

        for t in range(blks):
            blk = base + t * B
            pltpu.sync_copy(nbr.at[pl.ds(blk * deg // ICH, nch)], idx_v)
            pltpu.sync_copy(prior0.at[pl.ds(blk, B)], prior_v)
            pltpu.sync_copy(lp.at[pl.ds(blk, B)], lp_v)
            pltpu.sync_copy(l1p.at[pl.ds(blk, B)], l1p_v)
            pltpu.sync_copy(mist.at[pl.ds(blk, B)], mist_v)

            def fire(j, _):
                pltpu.async_copy(
                    tbl.at[idx_v.at[j]], mail_v.at[pl.ds(j * ICH, ICH)], sem)
                return 0

            def drain(j, _):
                pltpu.make_async_copy(
                    tbl.at[idx_v.at[j]], mail_v.at[pl.ds(j * ICH, ICH)], sem
                ).wait()
                return 0

            lax.fori_loop(0, nch, fire, 0)
            lax.fori_loop(0, nch, drain, 0)

            def group(g, _):
                prior = prior_v[pl.ds(g * L, L)]
                lpv = lp_v[pl.ds(g * L, L)]
                l1pv = l1p_v[pl.ds(g * L, L)]
                mv = mist_v[pl.ds(g * L, L)]
                rows0 = g * (L * deg) + lanes * deg
                for i in range(deg):
                    rows = rows0 + i
                    b = plsc.load_gather(mail_v, [rows, col_b])
                    s = plsc.load_gather(mail_v, [rows, col_s])
                    tt = plsc.load_gather(mail_v, [rows, col_t])
                    f = tt - s
                    valid = tt > 0.0
                    delta = jnp.abs(prior - b)
                    likely = jnp.exp(s * lpv + f * l1pv)
                    other = jnp.exp(s * l1pv + f * lpv)
                    p_l = prior * likely
                    marginal = p_l + other - prior * other
                    bel = p_l / marginal
                    omm = 1.0 - marginal
                    omm_g = jnp.where(valid, omm, 1.0)
                    misbel = (prior - p_l) / omm_g
                    certainty = 1.0 - jnp.minimum(delta * mv, 1.0) * omm
                    posterior = bel * certainty + misbel - misbel * certainty
                    prior = jnp.where(valid, posterior, prior)
                out_v[pl.ds(g * L, L)] = prior
                return 0

            lax.fori_loop(0, grp_per_blk, group, 0)
            pltpu.sync_copy(out_v, out.at[pl.ds(blk, B)])

    return sc_combine


def kernel(belief, probability, payoff, mistrust, neighbors):
    n = belief.shape[0]
    deg = neighbors.shape[1]
    c_per_w = -(-(-(-n // NW)) // B) * B
    n_pad = NW * c_per_w
    blks = c_per_w // B
    pad = n_pad - n

    f32 = jnp.float32
    belief_p = jnp.concatenate([belief, jnp.full((pad,), 0.5, f32)])
    prob_p = jnp.concatenate([probability, jnp.full((pad,), 0.5, f32)])
    mist_p = jnp.concatenate([mistrust, jnp.zeros((pad,), f32)])
    pad_idx = (jnp.arange(pad * deg, dtype=jnp.int32) % n).reshape(pad, deg)
    nbr_2d = jnp.concatenate([neighbors, pad_idx]).reshape(-1, ICH)
    tbl = jnp.concatenate(
        [belief[:, None], payoff, jnp.zeros((n, W - 3), f32)], axis=1)

    rows = n_pad // 128
    lp2, l1p2 = pl.pallas_call(
        _log_tables_kernel,
        out_shape=(jax.ShapeDtypeStruct((rows, 128), f32),
                   jax.ShapeDtypeStruct((rows, 128), f32)),
    )(prob_p.reshape(rows, 128))

    out = _make_sc_combine(n_pad, deg, c_per_w, blks)(
        tbl, nbr_2d, belief_p, lp2.reshape(n_pad), l1p2.reshape(n_pad),
        mist_p)
    return out[:n]


# speedup vs baseline: 94.2347x; 1.1404x over previous
import functools

import jax
import jax.numpy as jnp
from jax import lax
from jax.experimental import pallas as pl
from jax.experimental.pallas import tpu as pltpu
from jax.experimental.pallas import tpu_sc as plsc

NW = 32
L = 16
B = 224
W = 16
ICH = 128


def _log_tables_kernel(p_ref, lp_ref, l1p_ref):
    p = p_ref[...]
    lp_ref[...] = jnp.log(p)
    l1p_ref[...] = jnp.log(1.0 - p)


def _make_sc_combine(n_pad, deg, c_per_w, blks):
    mesh = plsc.VectorSubcoreMesh(
        core_axis_name="c", subcore_axis_name="s", num_cores=2,
        num_subcores=16)
    grp_per_blk = B // L
    nch = B * deg // ICH

    @functools.partial(
        pl.kernel,
        out_type=jax.ShapeDtypeStruct((n_pad,), jnp.float32),
        mesh=mesh,
        scratch_types=[
            pltpu.VMEM((2, nch, ICH), jnp.int32),
            pltpu.VMEM((2, B * deg, W), jnp.float32),
            pltpu.VMEM((2, B), jnp.float32),
            pltpu.VMEM((2, B), jnp.float32),
            pltpu.VMEM((2, B), jnp.float32),
            pltpu.VMEM((2, B), jnp.float32),
            pltpu.VMEM((2, B), jnp.float32),
            pltpu.SemaphoreType.DMA((2,)),
        ],
        compiler_params=pltpu.CompilerParams(
            needs_layout_passes=False, use_tc_tiling_on_sc=False),
    )
    def sc_combine(tbl, nbr, prior0, lp, l1p, mist, out,
                   idx_v, mail_v, prior_v, lp_v, l1p_v, mist_v, out_v, sem):
        wid = lax.axis_index("s") * 2 + lax.axis_index("c")
        base = wid * c_per_w
        lanes = lax.iota(jnp.int32, 16)
        col_b = jnp.full((16,), 0, jnp.int32)
        col_s = jnp.full((16,), 1, jnp.int32)
        col_t = jnp.full((16,), 2, jnp.int32)

        def stage(t, buf):
            blk = base + t * B
            pltpu.sync_copy(nbr.at[pl.ds(blk * deg // ICH, nch)],
                            idx_v.at[buf])
            pltpu.sync_copy(prior0.at[pl.ds(blk, B)], prior_v.at[buf])
            pltpu.sync_copy(lp.at[pl.ds(blk, B)], lp_v.at[buf])
            pltpu.sync_copy(l1p.at[pl.ds(blk, B)], l1p_v.at[buf])
            pltpu.sync_copy(mist.at[pl.ds(blk, B)], mist_v.at[buf])

            def fire(j, _):
                pltpu.async_copy(tbl.at[idx_v.at[buf, j]],
                                 mail_v.at[buf, pl.ds(j * ICH, ICH)],
                                 sem.at[buf])
                return 0

            lax.fori_loop(0, nch, fire, 0)

        stage(0, 0)

        def block(t, _):
            p = lax.rem(t, 2)
            q = lax.rem(t + 1, 2)

            @pl.when(t + 1 < blks)
            def _():
                stage(t + 1, q)

            def drain(j, _):
                pltpu.make_async_copy(tbl.at[idx_v.at[p, j]],
                                      mail_v.at[p, pl.ds(j * ICH, ICH)],
                                      sem.at[p]).wait()
                return 0

            lax.fori_loop(0, nch, drain, 0)
            mail_p = mail_v.at[p]

            def group(g, _):
                prior = prior_v.at[p][pl.ds(g * L, L)]
                lpv = lp_v.at[p][pl.ds(g * L, L)]
                l1pv = l1p_v.at[p][pl.ds(g * L, L)]
                mv = mist_v.at[p][pl.ds(g * L, L)]
                rows0 = g * (L * deg) + lanes * deg
                for i in range(deg):
                    rows = rows0 + i
                    b = plsc.load_gather(mail_p, [rows, col_b])
                    s = plsc.load_gather(mail_p, [rows, col_s])
                    tt = plsc.load_gather(mail_p, [rows, col_t])
                    f = tt - s
                    valid = tt > 0.0
                    delta = jnp.abs(prior - b)
                    likely = jnp.exp(s * lpv + f * l1pv)
                    other = jnp.exp(s * l1pv + f * lpv)
                    p_l = prior * likely
                    marginal = p_l + other - prior * other
                    bel = p_l / marginal
                    omm = 1.0 - marginal
                    omm_g = jnp.where(valid, omm, 1.0)
                    misbel = (prior - p_l) / omm_g
                    certainty = 1.0 - jnp.minimum(delta * mv, 1.0) * omm
                    posterior = bel * certainty + misbel - misbel * certainty
                    prior = jnp.where(valid, posterior, prior)
                out_v.at[p][pl.ds(g * L, L)] = prior
                return 0

            lax.fori_loop(0, grp_per_blk, group, 0)
            pltpu.sync_copy(out_v.at[p], out.at[pl.ds(base + t * B, B)])
            return 0

        lax.fori_loop(0, blks, block, 0)

    return sc_combine


def kernel(belief, probability, payoff, mistrust, neighbors):
    n = belief.shape[0]
    deg = neighbors.shape[1]
    c_per_w = -(-(-(-n // NW)) // B) * B
    n_pad = NW * c_per_w
    blks = c_per_w // B
    pad = n_pad - n

    f32 = jnp.float32
    belief_p = jnp.concatenate([belief, jnp.full((pad,), 0.5, f32)])
    prob_p = jnp.concatenate([probability, jnp.full((pad,), 0.5, f32)])
    mist_p = jnp.concatenate([mistrust, jnp.zeros((pad,), f32)])
    pad_idx = (jnp.arange(pad * deg, dtype=jnp.int32) % n).reshape(pad, deg)
    nbr_2d = jnp.concatenate([neighbors, pad_idx]).reshape(-1, ICH)
    tbl = jnp.concatenate(
        [belief[:, None], payoff, jnp.zeros((n, W - 3), f32)], axis=1)

    rows = n_pad // 128
    lp2, l1p2 = pl.pallas_call(
        _log_tables_kernel,
        out_shape=(jax.ShapeDtypeStruct((rows, 128), f32),
                   jax.ShapeDtypeStruct((rows, 128), f32)),
    )(prob_p.reshape(rows, 128))

    out = _make_sc_combine(n_pad, deg, c_per_w, blks)(
        tbl, nbr_2d, belief_p, lp2.reshape(n_pad), l1p2.reshape(n_pad),
        mist_p)
    return out[:n]


# speedup vs baseline: 101.6050x; 1.0782x over previous
import functools

import jax
import jax.numpy as jnp
from jax import lax
from jax.experimental import pallas as pl
from jax.experimental.pallas import tpu as pltpu
from jax.experimental.pallas import tpu_sc as plsc

NW = 32
L = 16
B = 224
W = 16
ICH = 128


def _log_tables_kernel(p_ref, lp_ref, l1p_ref):
    p = p_ref[...]
    lp_ref[...] = jnp.log(p)
    l1p_ref[...] = jnp.log(1.0 - p)


def _make_sc_combine(n_pad, deg, c_per_w, blks):
    mesh = plsc.VectorSubcoreMesh(
        core_axis_name="c", subcore_axis_name="s", num_cores=2,
        num_subcores=16)
    grp_per_blk = B // L
    nch = B * deg // ICH

    @functools.partial(
        pl.kernel,
        out_type=jax.ShapeDtypeStruct((n_pad,), jnp.float32),
        mesh=mesh,
        scratch_types=[
            pltpu.VMEM((2, nch, ICH), jnp.int32),
            pltpu.VMEM((2, B * deg, W), jnp.float32),
            pltpu.VMEM((2, B), jnp.float32),
            pltpu.VMEM((2, B), jnp.float32),
            pltpu.VMEM((2, B), jnp.float32),
            pltpu.VMEM((2, B), jnp.float32),
            pltpu.VMEM((2, B), jnp.float32),
            pltpu.SemaphoreType.DMA((2,)),
        ],
        compiler_params=pltpu.CompilerParams(
            needs_layout_passes=False, use_tc_tiling_on_sc=False),
    )
    def sc_combine(tbl, nbr, prior0, lp, l1p, mist, out,
                   idx_v, mail_v, prior_v, lp_v, l1p_v, mist_v, out_v, sem):
        wid = lax.axis_index("s") * 2 + lax.axis_index("c")
        base = wid * c_per_w
        lanes = lax.iota(jnp.int32, 16)
        col_b = jnp.full((16,), 0, jnp.int32)
        col_s = jnp.full((16,), 1, jnp.int32)
        col_t = jnp.full((16,), 2, jnp.int32)

        def stage(t, buf):
            blk = base + t * B
            pltpu.sync_copy(nbr.at[pl.ds(blk * deg // ICH, nch)],
                            idx_v.at[buf])
            pltpu.sync_copy(prior0.at[pl.ds(blk, B)], prior_v.at[buf])
            pltpu.sync_copy(lp.at[pl.ds(blk, B)], lp_v.at[buf])
            pltpu.sync_copy(l1p.at[pl.ds(blk, B)], l1p_v.at[buf])
            pltpu.sync_copy(mist.at[pl.ds(blk, B)], mist_v.at[buf])

            def fire(j, _):
                pltpu.async_copy(tbl.at[idx_v.at[buf, j]],
                                 mail_v.at[buf, pl.ds(j * ICH, ICH)],
                                 sem.at[buf])
                return 0

            lax.fori_loop(0, nch, fire, 0)

        stage(0, 0)

        def block(t, _):
            p = lax.rem(t, 2)
            q = lax.rem(t + 1, 2)

            @pl.when(t + 1 < blks)
            def _():
                stage(t + 1, q)

            def drain(j, _):
                pltpu.make_async_copy(tbl.at[idx_v.at[p, j]],
                                      mail_v.at[p, pl.ds(j * ICH, ICH)],
                                      sem.at[p]).wait()
                return 0

            lax.fori_loop(0, nch, drain, 0)
            mail_p = mail_v.at[p]

            def group(gh, _):
                gs = [gh * 2, gh * 2 + 1]
                prior = [prior_v.at[p][pl.ds(g * L, L)] for g in gs]
                lpv = [lp_v.at[p][pl.ds(g * L, L)] for g in gs]
                l1pv = [l1p_v.at[p][pl.ds(g * L, L)] for g in gs]
                mv = [mist_v.at[p][pl.ds(g * L, L)] for g in gs]
                rows0 = [g * (L * deg) + lanes * deg for g in gs]
                for i in range(deg):
                    for k in (0, 1):
                        rows = rows0[k] + i
                        b = plsc.load_gather(mail_p, [rows, col_b])
                        s = plsc.load_gather(mail_p, [rows, col_s])
                        tt = plsc.load_gather(mail_p, [rows, col_t])
                        pr = prior[k]
                        f = tt - s
                        valid = tt > 0.0
                        delta = jnp.abs(pr - b)
                        likely = jnp.exp(s * lpv[k] + f * l1pv[k])
                        other = jnp.exp(s * l1pv[k] + f * lpv[k])
                        p_l = pr * likely
                        marginal = p_l + other - pr * other
                        omm = 1.0 - marginal
                        omm_g = jnp.where(valid, omm, 1.0)
                        certainty = 1.0 - jnp.minimum(delta * mv[k], 1.0) * omm
                        num = (p_l * certainty * omm_g
                               + (pr - p_l) * (1.0 - certainty) * marginal)
                        posterior = num / (marginal * omm_g)
                        prior[k] = jnp.where(valid, posterior, pr)
                for k in (0, 1):
                    out_v.at[p][pl.ds(gs[k] * L, L)] = prior[k]
                return 0

            lax.fori_loop(0, grp_per_blk // 2, group, 0)
            pltpu.sync_copy(out_v.at[p], out.at[pl.ds(base + t * B, B)])
            return 0

        lax.fori_loop(0, blks, block, 0)

    return sc_combine


def kernel(belief, probability, payoff, mistrust, neighbors):
    n = belief.shape[0]
    deg = neighbors.shape[1]
    c_per_w = -(-(-(-n // NW)) // B) * B
    n_pad = NW * c_per_w
    blks = c_per_w // B
    pad = n_pad - n

    f32 = jnp.float32
    belief_p = jnp.concatenate([belief, jnp.full((pad,), 0.5, f32)])
    prob_p = jnp.concatenate([probability, jnp.full((pad,), 0.5, f32)])
    mist_p = jnp.concatenate([mistrust, jnp.zeros((pad,), f32)])
    pad_idx = (jnp.arange(pad * deg, dtype=jnp.int32) % n).reshape(pad, deg)
    nbr_2d = jnp.concatenate([neighbors, pad_idx]).reshape(-1, ICH)
    tbl = jnp.concatenate(
        [belief[:, None], payoff, jnp.zeros((n, W - 3), f32)], axis=1)

    rows = n_pad // 128
    lp2, l1p2 = pl.pallas_call(
        _log_tables_kernel,
        out_shape=(jax.ShapeDtypeStruct((rows, 128), f32),
                   jax.ShapeDtypeStruct((rows, 128), f32)),
    )(prob_p.reshape(rows, 128))

    out = _make_sc_combine(n_pad, deg, c_per_w, blks)(
        tbl, nbr_2d, belief_p, lp2.reshape(n_pad), l1p2.reshape(n_pad),
        mist_p)
    return out[:n]


# speedup vs baseline: 112.4651x; 1.1069x over previous
import functools

import jax
import jax.numpy as jnp
from jax import lax
from jax.experimental import pallas as pl
from jax.experimental.pallas import tpu as pltpu
from jax.experimental.pallas import tpu_sc as plsc

NW = 32
L = 16
B = 224
W = 16
ICH = 128


def _log_tables_kernel(p_ref, lp_ref, l1p_ref):
    p = p_ref[...]
    lp_ref[...] = jnp.log(p)
    l1p_ref[...] = jnp.log(1.0 - p)


def _make_sc_combine(n_pad, deg, c_per_w, blks):
    mesh = plsc.VectorSubcoreMesh(
        core_axis_name="c", subcore_axis_name="s", num_cores=2,
        num_subcores=16)
    grp_per_blk = B // L
    nch = B * deg // ICH

    @functools.partial(
        pl.kernel,
        out_type=jax.ShapeDtypeStruct((n_pad,), jnp.float32),
        mesh=mesh,
        scratch_types=[
            pltpu.VMEM((2, nch, ICH), jnp.int32),
            pltpu.VMEM((2, B * deg, W), jnp.float32),
            pltpu.VMEM((2, B), jnp.float32),
            pltpu.VMEM((2, B), jnp.float32),
            pltpu.VMEM((2, B), jnp.float32),
            pltpu.VMEM((2, B), jnp.float32),
            pltpu.VMEM((2, B), jnp.float32),
            pltpu.SemaphoreType.DMA((2,)),
            pltpu.SemaphoreType.DMA((2,)),
        ],
        compiler_params=pltpu.CompilerParams(
            needs_layout_passes=False, use_tc_tiling_on_sc=False),
    )
    def sc_combine(tbl, nbr, prior0, lp, l1p, mist, out,
                   idx_v, mail_v, prior_v, lp_v, l1p_v, mist_v, out_v, sem,
                   sem2):
        wid = lax.axis_index("s") * 2 + lax.axis_index("c")
        base = wid * c_per_w
        lanes = lax.iota(jnp.int32, 16)
        col_b = jnp.full((16,), 0, jnp.int32)
        col_s = jnp.full((16,), 1, jnp.int32)
        col_t = jnp.full((16,), 2, jnp.int32)

        def state_copies(t, buf):
            blk = base + t * B
            yield prior0.at[pl.ds(blk, B)], prior_v.at[buf]
            yield lp.at[pl.ds(blk, B)], lp_v.at[buf]
            yield l1p.at[pl.ds(blk, B)], l1p_v.at[buf]
            yield mist.at[pl.ds(blk, B)], mist_v.at[buf]

        def stage(t, buf):
            blk = base + t * B
            pltpu.sync_copy(nbr.at[pl.ds(blk * deg // ICH, nch)],
                            idx_v.at[buf])

            def fire(j, _):
                pltpu.async_copy(tbl.at[idx_v.at[buf, j]],
                                 mail_v.at[buf, pl.ds(j * ICH, ICH)],
                                 sem.at[buf])
                return 0

            lax.fori_loop(0, nch, fire, 0)
            for src, dst in state_copies(t, buf):
                pltpu.async_copy(src, dst, sem2.at[buf])

        stage(0, 0)

        def block(t, _):
            p = lax.rem(t, 2)
            q = lax.rem(t + 1, 2)

            @pl.when(t + 1 < blks)
            def _():
                stage(t + 1, q)

            def drain(j, _):
                pltpu.make_async_copy(tbl.at[idx_v.at[p, j]],
                                      mail_v.at[p, pl.ds(j * ICH, ICH)],
                                      sem.at[p]).wait()
                return 0

            lax.fori_loop(0, nch, drain, 0)
            for src, dst in state_copies(t, p):
                pltpu.make_async_copy(src, dst, sem2.at[p]).wait()
            mail_p = mail_v.at[p]

            def group(gh, _):
                gs = [gh * 2, gh * 2 + 1]
                prior = [prior_v.at[p][pl.ds(g * L, L)] for g in gs]
                lpv = [lp_v.at[p][pl.ds(g * L, L)] for g in gs]
                l1pv = [l1p_v.at[p][pl.ds(g * L, L)] for g in gs]
                mv = [mist_v.at[p][pl.ds(g * L, L)] for g in gs]
                rows0 = [g * (L * deg) + lanes * deg for g in gs]
                for i in range(deg):
                    for k in (0, 1):
                        rows = rows0[k] + i
                        b = plsc.load_gather(mail_p, [rows, col_b])
                        s = plsc.load_gather(mail_p, [rows, col_s])
                        tt = plsc.load_gather(mail_p, [rows, col_t])
                        pr = prior[k]
                        f = tt - s
                        valid = tt > 0.0
                        delta = jnp.abs(pr - b)
                        likely = jnp.exp(s * lpv[k] + f * l1pv[k])
                        other = jnp.exp(s * l1pv[k] + f * lpv[k])
                        p_l = pr * likely
                        marginal = p_l + other - pr * other
                        omm = 1.0 - marginal
                        omm_g = jnp.where(valid, omm, 1.0)
                        certainty = 1.0 - jnp.minimum(delta * mv[k], 1.0) * omm
                        num = (p_l * certainty * omm_g
                               + (pr - p_l) * (1.0 - certainty) * marginal)
                        posterior = num / (marginal * omm_g)
                        prior[k] = jnp.where(valid, posterior, pr)
                for k in (0, 1):
                    out_v.at[p][pl.ds(gs[k] * L, L)] = prior[k]
                return 0

            lax.fori_loop(0, grp_per_blk // 2, group, 0)
            pltpu.sync_copy(out_v.at[p], out.at[pl.ds(base + t * B, B)])
            return 0

        lax.fori_loop(0, blks, block, 0)

    return sc_combine


def kernel(belief, probability, payoff, mistrust, neighbors):
    n = belief.shape[0]
    deg = neighbors.shape[1]
    c_per_w = -(-(-(-n // NW)) // B) * B
    n_pad = NW * c_per_w
    blks = c_per_w // B
    pad = n_pad - n

    f32 = jnp.float32
    belief_p = jnp.concatenate([belief, jnp.full((pad,), 0.5, f32)])
    prob_p = jnp.concatenate([probability, jnp.full((pad,), 0.5, f32)])
    mist_p = jnp.concatenate([mistrust, jnp.zeros((pad,), f32)])
    pad_idx = (jnp.arange(pad * deg, dtype=jnp.int32) % n).reshape(pad, deg)
    nbr_2d = jnp.concatenate([neighbors, pad_idx]).reshape(-1, ICH)
    tbl = jnp.concatenate(
        [belief[:, None], payoff, jnp.zeros((n, W - 3), f32)], axis=1)

    rows = n_pad // 128
    lp2, l1p2 = pl.pallas_call(
        _log_tables_kernel,
        out_shape=(jax.ShapeDtypeStruct((rows, 128), f32),
                   jax.ShapeDtypeStruct((rows, 128), f32)),
    )(prob_p.reshape(rows, 128))

    out = _make_sc_combine(n_pad, deg, c_per_w, blks)(
        tbl, nbr_2d, belief_p, lp2.reshape(n_pad), l1p2.reshape(n_pad),
        mist_p)
    return out[:n]


# speedup vs baseline: 117.6007x; 1.0457x over previous
import functools

import jax
import jax.numpy as jnp
from jax import lax
from jax.experimental import pallas as pl
from jax.experimental.pallas import tpu as pltpu
from jax.experimental.pallas import tpu_sc as plsc

NW = 32
L = 16
B = 224
W = 16
ICH = 128


def _log_tables_kernel(p_ref, lp_ref, l1p_ref):
    p = p_ref[...]
    lp_ref[...] = jnp.log(p)
    l1p_ref[...] = jnp.log(1.0 - p)


def _make_sc_combine(n_pad, deg, c_per_w, blks):
    mesh = plsc.VectorSubcoreMesh(
        core_axis_name="c", subcore_axis_name="s", num_cores=2,
        num_subcores=16)
    grp_per_blk = B // L
    nch = B * deg // ICH

    @functools.partial(
        pl.kernel,
        out_type=jax.ShapeDtypeStruct((n_pad,), jnp.float32),
        mesh=mesh,
        scratch_types=[
            pltpu.VMEM((2, nch, ICH), jnp.int32),
            pltpu.VMEM((2, B * deg, W), jnp.float32),
            pltpu.VMEM((2, B), jnp.float32),
            pltpu.VMEM((2, B), jnp.float32),
            pltpu.VMEM((2, B), jnp.float32),
            pltpu.VMEM((2, B), jnp.float32),
            pltpu.VMEM((2, B), jnp.float32),
            pltpu.SemaphoreType.DMA((2,)),
            pltpu.SemaphoreType.DMA((2,)),
            pltpu.SemaphoreType.DMA((2,)),
            pltpu.SemaphoreType.DMA((2,)),
        ],
        compiler_params=pltpu.CompilerParams(
            needs_layout_passes=False, use_tc_tiling_on_sc=False),
    )
    def sc_combine(tbl, nbr, prior0, lp, l1p, mist, out,
                   idx_v, mail_v, prior_v, lp_v, l1p_v, mist_v, out_v, sem,
                   sem2, sem3, sem4):
        wid = lax.axis_index("s") * 2 + lax.axis_index("c")
        base = wid * c_per_w
        lanes = lax.iota(jnp.int32, 16)
        col_b = jnp.full((16,), 0, jnp.int32)
        col_s = jnp.full((16,), 1, jnp.int32)
        col_t = jnp.full((16,), 2, jnp.int32)

        def state_copies(t, buf):
            blk = base + t * B
            yield prior0.at[pl.ds(blk, B)], prior_v.at[buf]
            yield lp.at[pl.ds(blk, B)], lp_v.at[buf]
            yield l1p.at[pl.ds(blk, B)], l1p_v.at[buf]
            yield mist.at[pl.ds(blk, B)], mist_v.at[buf]

        def idx_copy(t, buf):
            blk = base + t * B
            return (nbr.at[pl.ds(blk * deg // ICH, nch)], idx_v.at[buf])

        def fire_block(t, buf):
            def fire(j, _):
                pltpu.async_copy(tbl.at[idx_v.at[buf, j]],
                                 mail_v.at[buf, pl.ds(j * ICH, ICH)],
                                 sem.at[buf])
                return 0

            lax.fori_loop(0, nch, fire, 0)
            for src, dst in state_copies(t, buf):
                pltpu.async_copy(src, dst, sem2.at[buf])

        pltpu.sync_copy(*idx_copy(0, 0))
        fire_block(0, 0)

        @pl.when(blks > 1)
        def _():
            src, dst = idx_copy(1, 1)
            pltpu.async_copy(src, dst, sem3.at[1])

        def block(t, _):
            p = lax.rem(t, 2)
            q = lax.rem(t + 1, 2)

            @pl.when(t + 1 < blks)
            def _():
                src, dst = idx_copy(t + 1, q)
                pltpu.make_async_copy(src, dst, sem3.at[q]).wait()
                fire_block(t + 1, q)

            def drain(j, _):
                pltpu.make_async_copy(tbl.at[idx_v.at[p, j]],
                                      mail_v.at[p, pl.ds(j * ICH, ICH)],
                                      sem.at[p]).wait()
                return 0

            lax.fori_loop(0, nch, drain, 0)

            @pl.when(t + 2 < blks)
            def _():
                src, dst = idx_copy(t + 2, p)
                pltpu.async_copy(src, dst, sem3.at[p])

            @pl.when(t >= 2)
            def _():
                pltpu.make_async_copy(
                    out_v.at[p], out.at[pl.ds(base + (t - 2) * B, B)],
                    sem4.at[p]).wait()

            for src, dst in state_copies(t, p):
                pltpu.make_async_copy(src, dst, sem2.at[p]).wait()
            mail_p = mail_v.at[p]

            def group(gh, _):
                gs = [gh * 2, gh * 2 + 1]
                prior = [prior_v.at[p][pl.ds(g * L, L)] for g in gs]
                lpv = [lp_v.at[p][pl.ds(g * L, L)] for g in gs]
                l1pv = [l1p_v.at[p][pl.ds(g * L, L)] for g in gs]
                mv = [mist_v.at[p][pl.ds(g * L, L)] for g in gs]
                rows0 = [g * (L * deg) + lanes * deg for g in gs]
                for i in range(deg):
                    for k in (0, 1):
                        rows = rows0[k] + i
                        b = plsc.load_gather(mail_p, [rows, col_b])
                        s = plsc.load_gather(mail_p, [rows, col_s])
                        tt = plsc.load_gather(mail_p, [rows, col_t])
                        pr = prior[k]
                        f = tt - s
                        valid = tt > 0.0
                        delta = jnp.abs(pr - b)
                        likely = jnp.exp(s * lpv[k] + f * l1pv[k])
                        other = jnp.exp(s * l1pv[k] + f * lpv[k])
                        p_l = pr * likely
                        marginal = p_l + other - pr * other
                        omm = 1.0 - marginal
                        omm_g = jnp.where(valid, omm, 1.0)
                        certainty = 1.0 - jnp.minimum(delta * mv[k], 1.0) * omm
                        num = (p_l * certainty * omm_g
                               + (pr - p_l) * (1.0 - certainty) * marginal)
                        posterior = num / (marginal * omm_g)
                        prior[k] = jnp.where(valid, posterior, pr)
                for k in (0, 1):
                    out_v.at[p][pl.ds(gs[k] * L, L)] = prior[k]
                return 0

            lax.fori_loop(0, grp_per_blk // 2, group, 0)
            pltpu.async_copy(out_v.at[p], out.at[pl.ds(base + t * B, B)],
                             sem4.at[p])
            return 0

        lax.fori_loop(0, blks, block, 0)
        for t in range(max(0, blks - 2), blks):
            pltpu.make_async_copy(
                out_v.at[t % 2], out.at[pl.ds(base + t * B, B)],
                sem4.at[t % 2]).wait()

    return sc_combine


def kernel(belief, probability, payoff, mistrust, neighbors):
    n = belief.shape[0]
    deg = neighbors.shape[1]
    c_per_w = -(-(-(-n // NW)) // B) * B
    n_pad = NW * c_per_w
    blks = c_per_w // B
    pad = n_pad - n

    f32 = jnp.float32
    belief_p = jnp.concatenate([belief, jnp.full((pad,), 0.5, f32)])
    prob_p = jnp.concatenate([probability, jnp.full((pad,), 0.5, f32)])
    mist_p = jnp.concatenate([mistrust, jnp.zeros((pad,), f32)])
    pad_idx = (jnp.arange(pad * deg, dtype=jnp.int32) % n).reshape(pad, deg)
    nbr_2d = jnp.concatenate([neighbors, pad_idx]).reshape(-1, ICH)
    tbl = jnp.concatenate(
        [belief[:, None], payoff, jnp.zeros((n, W - 3), f32)], axis=1)

    rows = n_pad // 128
    lp2, l1p2 = pl.pallas_call(
        _log_tables_kernel,
        out_shape=(jax.ShapeDtypeStruct((rows, 128), f32),
                   jax.ShapeDtypeStruct((rows, 128), f32)),
    )(prob_p.reshape(rows, 128))

    out = _make_sc_combine(n_pad, deg, c_per_w, blks)(
        tbl, nbr_2d, belief_p, lp2.reshape(n_pad), l1p2.reshape(n_pad),
        mist_p)
    return out[:n]


# speedup vs baseline: 118.3944x; 1.0067x over previous
import functools

import jax
import jax.numpy as jnp
from jax import lax
from jax.experimental import pallas as pl
from jax.experimental.pallas import tpu as pltpu
from jax.experimental.pallas import tpu_sc as plsc

NW = 32
L = 16
B = 224
W = 16
ICH = 128


def _log_tables_kernel(p_ref, lp_ref, l1p_ref):
    p = p_ref[...]
    lp_ref[...] = jnp.log(p)
    l1p_ref[...] = jnp.log(1.0 - p)


def _make_sc_combine(n_pad, deg, c_per_w, blks):
    mesh = plsc.VectorSubcoreMesh(
        core_axis_name="c", subcore_axis_name="s", num_cores=2,
        num_subcores=16)
    grp_per_blk = B // L
    nch = B * deg // ICH

    @functools.partial(
        pl.kernel,
        out_type=jax.ShapeDtypeStruct((n_pad,), jnp.float32),
        mesh=mesh,
        scratch_types=[
            pltpu.VMEM((2, nch, ICH), jnp.int32),
            pltpu.VMEM((2, B * deg, W), jnp.float32),
            pltpu.VMEM((2, B), jnp.float32),
            pltpu.VMEM((2, B), jnp.float32),
            pltpu.VMEM((2, B), jnp.float32),
            pltpu.VMEM((2, B), jnp.float32),
            pltpu.VMEM((2, B), jnp.float32),
            pltpu.SemaphoreType.DMA((2,)),
            pltpu.SemaphoreType.DMA((2,)),
            pltpu.SemaphoreType.DMA((2,)),
            pltpu.SemaphoreType.DMA((2,)),
        ],
        compiler_params=pltpu.CompilerParams(
            needs_layout_passes=False, use_tc_tiling_on_sc=False),
    )
    def sc_combine(tbl, nbr, prior0, lp, l1p, mist, out,
                   idx_v, mail_v, prior_v, lp_v, l1p_v, mist_v, out_v, sem,
                   sem2, sem3, sem4):
        wid = lax.axis_index("s") * 2 + lax.axis_index("c")
        base = wid * c_per_w
        lanes = lax.iota(jnp.int32, 16)
        col_b = jnp.full((16,), 0, jnp.int32)
        col_s = jnp.full((16,), 1, jnp.int32)
        col_t = jnp.full((16,), 2, jnp.int32)

        def state_copies(t, buf):
            blk = base + t * B
            yield prior0.at[pl.ds(blk, B)], prior_v.at[buf]
            yield lp.at[pl.ds(blk, B)], lp_v.at[buf]
            yield l1p.at[pl.ds(blk, B)], l1p_v.at[buf]
            yield mist.at[pl.ds(blk, B)], mist_v.at[buf]

        def idx_copy(t, buf):
            blk = base + t * B
            return (nbr.at[pl.ds(blk * deg // ICH, nch)], idx_v.at[buf])

        def fire_block(t, buf):
            def fire(j, _):
                pltpu.async_copy(tbl.at[idx_v.at[buf, j]],
                                 mail_v.at[buf, pl.ds(j * ICH, ICH)],
                                 sem.at[buf])
                return 0

            lax.fori_loop(0, nch, fire, 0)
            for src, dst in state_copies(t, buf):
                pltpu.async_copy(src, dst, sem2.at[buf])

        pltpu.sync_copy(*idx_copy(0, 0))
        fire_block(0, 0)

        @pl.when(blks > 1)
        def _():
            src, dst = idx_copy(1, 1)
            pltpu.async_copy(src, dst, sem3.at[1])

        def block(t, _):
            p = lax.rem(t, 2)
            q = lax.rem(t + 1, 2)

            @pl.when(t + 1 < blks)
            def _():
                src, dst = idx_copy(t + 1, q)
                pltpu.make_async_copy(src, dst, sem3.at[q]).wait()
                fire_block(t + 1, q)

            pltpu.make_async_copy(tbl.at[pl.ds(0, B * deg)], mail_v.at[p],
                                  sem.at[p]).wait()

            @pl.when(t + 2 < blks)
            def _():
                src, dst = idx_copy(t + 2, p)
                pltpu.async_copy(src, dst, sem3.at[p])

            @pl.when(t >= 2)
            def _():
                pltpu.make_async_copy(
                    out_v.at[p], out.at[pl.ds(base + (t - 2) * B, B)],
                    sem4.at[p]).wait()

            for src, dst in state_copies(t, p):
                pltpu.make_async_copy(src, dst, sem2.at[p]).wait()
            mail_p = mail_v.at[p]

            def group(gh, _):
                gs = [gh * 2, gh * 2 + 1]
                prior = [prior_v.at[p][pl.ds(g * L, L)] for g in gs]
                lpv = [lp_v.at[p][pl.ds(g * L, L)] for g in gs]
                l1pv = [l1p_v.at[p][pl.ds(g * L, L)] for g in gs]
                mv = [mist_v.at[p][pl.ds(g * L, L)] for g in gs]
                rows0 = [g * (L * deg) + lanes * deg for g in gs]
                for i in range(deg):
                    for k in (0, 1):
                        rows = rows0[k] + i
                        b = plsc.load_gather(mail_p, [rows, col_b])
                        s = plsc.load_gather(mail_p, [rows, col_s])
                        tt = plsc.load_gather(mail_p, [rows, col_t])
                        pr = prior[k]
                        f = tt - s
                        valid = tt > 0.0
                        delta = jnp.abs(pr - b)
                        likely = jnp.exp(s * lpv[k] + f * l1pv[k])
                        other = jnp.exp(s * l1pv[k] + f * lpv[k])
                        p_l = pr * likely
                        marginal = p_l + other - pr * other
                        omm = 1.0 - marginal
                        omm_g = jnp.where(valid, omm, 1.0)
                        certainty = 1.0 - jnp.minimum(delta * mv[k], 1.0) * omm
                        num = (p_l * certainty * omm_g
                               + (pr - p_l) * (1.0 - certainty) * marginal)
                        posterior = num / (marginal * omm_g)
                        prior[k] = jnp.where(valid, posterior, pr)
                for k in (0, 1):
                    out_v.at[p][pl.ds(gs[k] * L, L)] = prior[k]
                return 0

            lax.fori_loop(0, grp_per_blk // 2, group, 0)
            pltpu.async_copy(out_v.at[p], out.at[pl.ds(base + t * B, B)],
                             sem4.at[p])
            return 0

        lax.fori_loop(0, blks, block, 0)
        for t in range(max(0, blks - 2), blks):
            pltpu.make_async_copy(
                out_v.at[t % 2], out.at[pl.ds(base + t * B, B)],
                sem4.at[t % 2]).wait()

    return sc_combine


def kernel(belief, probability, payoff, mistrust, neighbors):
    n = belief.shape[0]
    deg = neighbors.shape[1]
    c_per_w = -(-(-(-n // NW)) // B) * B
    n_pad = NW * c_per_w
    blks = c_per_w // B
    pad = n_pad - n

    f32 = jnp.float32
    belief_p = jnp.concatenate([belief, jnp.full((pad,), 0.5, f32)])
    prob_p = jnp.concatenate([probability, jnp.full((pad,), 0.5, f32)])
    mist_p = jnp.concatenate([mistrust, jnp.zeros((pad,), f32)])
    pad_idx = (jnp.arange(pad * deg, dtype=jnp.int32) % n).reshape(pad, deg)
    nbr_2d = jnp.concatenate([neighbors, pad_idx]).reshape(-1, ICH)
    tbl = jnp.concatenate(
        [belief[:, None], payoff, jnp.zeros((n, W - 3), f32)], axis=1)

    rows = n_pad // 128
    lp2, l1p2 = pl.pallas_call(
        _log_tables_kernel,
        out_shape=(jax.ShapeDtypeStruct((rows, 128), f32),
                   jax.ShapeDtypeStruct((rows, 128), f32)),
    )(prob_p.reshape(rows, 128))

    out = _make_sc_combine(n_pad, deg, c_per_w, blks)(
        tbl, nbr_2d, belief_p, lp2.reshape(n_pad), l1p2.reshape(n_pad),
        mist_p)
    return out[:n]


# speedup vs baseline: 119.9714x; 1.0133x over previous
import functools

import jax
import jax.numpy as jnp
from jax import lax
from jax.experimental import pallas as pl
from jax.experimental.pallas import tpu as pltpu
from jax.experimental.pallas import tpu_sc as plsc

NW = 32
L = 16
B = 224
W = 16
ICH = 128


def _log_tables_kernel(p_ref, lp_ref, l1p_ref):
    p = p_ref[...]
    lp_ref[...] = jnp.log(p)
    l1p_ref[...] = jnp.log(1.0 - p)


def _make_sc_combine(n, deg, c_per_w, blks):
    mesh = plsc.VectorSubcoreMesh(
        core_axis_name="c", subcore_axis_name="s", num_cores=2,
        num_subcores=16)
    grp_per_blk = B // L
    nch = B * deg // ICH

    @functools.partial(
        pl.kernel,
        out_type=jax.ShapeDtypeStruct((n,), jnp.float32),
        mesh=mesh,
        scratch_types=[
            pltpu.VMEM((2, nch, ICH), jnp.int32),
            pltpu.VMEM((2, B * deg, W), jnp.float32),
            pltpu.VMEM((2, B), jnp.float32),
            pltpu.VMEM((2, B), jnp.float32),
            pltpu.VMEM((2, B), jnp.float32),
            pltpu.VMEM((2, B), jnp.float32),
            pltpu.VMEM((2, B), jnp.float32),
            pltpu.SemaphoreType.DMA((2,)),
            pltpu.SemaphoreType.DMA((2,)),
            pltpu.SemaphoreType.DMA((2,)),
            pltpu.SemaphoreType.DMA((2,)),
        ],
        compiler_params=pltpu.CompilerParams(
            needs_layout_passes=False, use_tc_tiling_on_sc=False),
    )
    def sc_combine(tbl, nbr, prior0, lp, l1p, mist, out,
                   idx_v, mail_v, prior_v, lp_v, l1p_v, mist_v, out_v, sem,
                   sem2, sem3, sem4):
        wid = lax.axis_index("s") * 2 + lax.axis_index("c")
        base = jnp.minimum(wid * c_per_w, n - c_per_w)
        lanes = lax.iota(jnp.int32, 16)
        col_b = jnp.full((16,), 0, jnp.int32)
        col_s = jnp.full((16,), 1, jnp.int32)
        col_t = jnp.full((16,), 2, jnp.int32)

        def state_copies(t, buf):
            blk = base + t * B
            yield prior0.at[pl.ds(blk, B)], prior_v.at[buf]
            yield lp.at[pl.ds(blk, B)], lp_v.at[buf]
            yield l1p.at[pl.ds(blk, B)], l1p_v.at[buf]
            yield mist.at[pl.ds(blk, B)], mist_v.at[buf]

        def idx_copy(t, buf):
            blk = base + t * B
            return (nbr.at[pl.ds(blk * deg // ICH, nch)], idx_v.at[buf])

        def fire_block(t, buf):
            def fire(j, _):
                pltpu.async_copy(tbl.at[idx_v.at[buf, j]],
                                 mail_v.at[buf, pl.ds(j * ICH, ICH)],
                                 sem.at[buf])
                return 0

            lax.fori_loop(0, nch, fire, 0)
            for src, dst in state_copies(t, buf):
                pltpu.async_copy(src, dst, sem2.at[buf])

        pltpu.sync_copy(*idx_copy(0, 0))
        fire_block(0, 0)

        @pl.when(blks > 1)
        def _():
            src, dst = idx_copy(1, 1)
            pltpu.async_copy(src, dst, sem3.at[1])

        def block(t, _):
            p = lax.rem(t, 2)
            q = lax.rem(t + 1, 2)

            @pl.when(t + 1 < blks)
            def _():
                src, dst = idx_copy(t + 1, q)
                pltpu.make_async_copy(src, dst, sem3.at[q]).wait()
                fire_block(t + 1, q)

            pltpu.make_async_copy(tbl.at[pl.ds(0, B * deg)], mail_v.at[p],
                                  sem.at[p]).wait()

            @pl.when(t + 2 < blks)
            def _():
                src, dst = idx_copy(t + 2, p)
                pltpu.async_copy(src, dst, sem3.at[p])

            @pl.when(t >= 2)
            def _():
                pltpu.make_async_copy(
                    out_v.at[p], out.at[pl.ds(base + (t - 2) * B, B)],
                    sem4.at[p]).wait()

            for src, dst in state_copies(t, p):
                pltpu.make_async_copy(src, dst, sem2.at[p]).wait()
            mail_p = mail_v.at[p]

            def group(gh, _):
                gs = [gh * 2, gh * 2 + 1]
                prior = [prior_v.at[p][pl.ds(g * L, L)] for g in gs]
                lpv = [lp_v.at[p][pl.ds(g * L, L)] for g in gs]
                l1pv = [l1p_v.at[p][pl.ds(g * L, L)] for g in gs]
                mv = [mist_v.at[p][pl.ds(g * L, L)] for g in gs]
                rows0 = [g * (L * deg) + lanes * deg for g in gs]
                for i in range(deg):
                    for k in (0, 1):
                        rows = rows0[k] + i
                        b = plsc.load_gather(mail_p, [rows, col_b])
                        s = plsc.load_gather(mail_p, [rows, col_s])
                        tt = plsc.load_gather(mail_p, [rows, col_t])
                        pr = prior[k]
                        f = tt - s
                        valid = tt > 0.0
                        delta = jnp.abs(pr - b)
                        likely = jnp.exp(s * lpv[k] + f * l1pv[k])
                        other = jnp.exp(s * l1pv[k] + f * lpv[k])
                        p_l = pr * likely
                        marginal = p_l + other - pr * other
                        omm = 1.0 - marginal
                        omm_g = jnp.where(valid, omm, 1.0)
                        certainty = 1.0 - jnp.minimum(delta * mv[k], 1.0) * omm
                        num = (p_l * certainty * omm_g
                               + (pr - p_l) * (1.0 - certainty) * marginal)
                        posterior = num / (marginal * omm_g)
                        prior[k] = jnp.where(valid, posterior, pr)
                for k in (0, 1):
                    out_v.at[p][pl.ds(gs[k] * L, L)] = prior[k]
                return 0

            lax.fori_loop(0, grp_per_blk // 2, group, 0)
            pltpu.async_copy(out_v.at[p], out.at[pl.ds(base + t * B, B)],
                             sem4.at[p])
            return 0

        lax.fori_loop(0, blks, block, 0)
        for t in range(max(0, blks - 2), blks):
            pltpu.make_async_copy(
                out_v.at[t % 2], out.at[pl.ds(base + t * B, B)],
                sem4.at[t % 2]).wait()

    return sc_combine


def kernel(belief, probability, payoff, mistrust, neighbors):
    n = belief.shape[0]
    deg = neighbors.shape[1]
    c_per_w = -(-(-(-n // NW)) // B) * B
    blks = c_per_w // B

    f32 = jnp.float32
    nbr_2d = neighbors.reshape(-1, ICH)
    tbl = jnp.concatenate(
        [belief[:, None], payoff, jnp.zeros((n, W - 3), f32)], axis=1)

    rows = 625 if n == 100000 else n // 128
    lp2, l1p2 = pl.pallas_call(
        _log_tables_kernel,
        out_shape=(jax.ShapeDtypeStruct((rows, n // rows), f32),
                   jax.ShapeDtypeStruct((rows, n // rows), f32)),
    )(probability.reshape(rows, n // rows))

    return _make_sc_combine(n, deg, c_per_w, blks)(
        tbl, nbr_2d, belief, lp2.reshape(n), l1p2.reshape(n), mistrust)
